# Initial kernel scaffold; baseline (speedup 1.0000x reference)
import functools

import jax
import jax.numpy as jnp
from jax.experimental import pallas as pl

_HI = jax.lax.Precision.HIGHEST
_BIG = jnp.float32(3.0e38)


def _dot(a, b):
    return jax.lax.dot_general(a, b, (((1,), (0,)), ((), ())),
                               precision=_HI, preferred_element_type=jnp.float32)


def _dist_matrix(qp, pct):
    qt = qp.shape[0]
    n = pct.shape[1]
    q2 = jnp.sum(qp * qp, axis=1, keepdims=True)
    p2 = jnp.sum(pct * pct, axis=0, keepdims=True)
    aq = jnp.concatenate([q2, jnp.ones((qt, 1), jnp.float32), qp], axis=1)
    bp = jnp.concatenate([jnp.ones((1, n), jnp.float32), p2, -2.0 * pct], axis=0)
    return _dot(aq, bp)


def _argmin_onehot(dist, iota):
    m = jnp.min(dist, axis=1, keepdims=True)
    cand = jnp.where(dist <= m, iota, jnp.int32(0x7FFFFFFF))
    am = jnp.min(cand, axis=1, keepdims=True)
    return m, iota == am


def _conv_kernel(s_ref, pct_ref, qp_ref, *wrefs, out_ref, n_layers, k_nn):
    layers = [(wrefs[3 * i][...], wrefs[3 * i + 1][...], wrefs[3 * i + 2][...])
              for i in range(n_layers)]
    s = s_ref[0]
    pct = pct_ref[0]
    qp = qp_ref[0]
    n = s.shape[0]
    qt = qp.shape[0]
    d_out = layers[-1][0].shape[1]

    dist = _dist_matrix(qp, pct)
    iota = jax.lax.broadcasted_iota(jnp.int32, (qt, n), 1)
    w1 = layers[0][0]
    qw = _dot(qp, w1[:3, :])

    def body(_, carry):
        dist, acc = carry
        _, onehot = _argmin_onehot(dist, iota)
        g = _dot(onehot.astype(jnp.float32), s)
        h = jnp.maximum((_dot(g, w1) - qw) * layers[0][1] + layers[0][2], 0.0)
        for w, ga, be in layers[1:]:
            h = jnp.maximum(_dot(h, w) * ga + be, 0.0)
        acc = jnp.maximum(acc, h)
        dist = jnp.where(onehot, _BIG, dist)
        return dist, acc

    _, acc = jax.lax.fori_loop(
        0, k_nn, body, (dist, jnp.zeros((qt, d_out), jnp.float32)))
    out_ref[0] = acc


def _conv_stage(s, pct, qp, layers, k_nn, q_tile):
    b, n, d = s.shape
    q = qp.shape[1]
    q_tile = min(q_tile, q)
    d_out = layers[-1][0].shape[1]
    wargs = []
    for (w, ga, be) in layers:
        wargs += [w, ga.reshape(1, -1), be.reshape(1, -1)]
    grid = (b, q // q_tile)
    full = lambda arr: pl.BlockSpec(arr.shape, lambda bi, qi: (0,) * arr.ndim)
    return pl.pallas_call(
        functools.partial(_conv_kernel, n_layers=len(layers), k_nn=k_nn),
        grid=grid,
        in_specs=[
            pl.BlockSpec((1, n, d), lambda bi, qi: (bi, 0, 0)),
            pl.BlockSpec((1, 3, n), lambda bi, qi: (bi, 0, 0)),
            pl.BlockSpec((1, q_tile, 3), lambda bi, qi: (bi, qi, 0)),
        ] + [full(a) for a in wargs],
        out_specs=pl.BlockSpec((1, q_tile, d_out), lambda bi, qi: (bi, qi, 0)),
        out_shape=jax.ShapeDtypeStruct((b, q, d_out), jnp.float32),
    )(s, pct, qp, *wargs)


def _deconv_kernel(fc_ref, pct_ref, qp_ref, *wrefs, out_ref, n_layers, has_ff):
    off = 0
    if has_ff:
        ff = wrefs[0][0]
        off = 1
    layers = [(wrefs[off + 3 * i][...], wrefs[off + 3 * i + 1][...],
               wrefs[off + 3 * i + 2][...]) for i in range(n_layers)]
    fc = fc_ref[0]
    pct = pct_ref[0]
    qp = qp_ref[0]
    nc, cc = fc.shape
    qt = qp.shape[0]

    dist = _dist_matrix(qp, pct)
    iota = jax.lax.broadcasted_iota(jnp.int32, (qt, nc), 1)
    num = jnp.zeros((qt, cc), jnp.float32)
    den = jnp.zeros((qt, 1), jnp.float32)
    for _ in range(3):
        m, onehot = _argmin_onehot(dist, iota)
        f = _dot(onehot.astype(jnp.float32), fc)
        w = 1.0 / jnp.maximum(m, 1e-10)
        num = num + w * f
        den = den + w
        dist = jnp.where(onehot, _BIG, dist)
    interp = num / den

    w1, g1, b1 = layers[0]
    pre = _dot(interp, w1[:cc, :])
    if has_ff:
        pre = pre + _dot(ff, w1[cc:, :])
    h = jnp.maximum(pre * g1 + b1, 0.0)
    for w, ga, be in layers[1:]:
        h = jnp.maximum(_dot(h, w) * ga + be, 0.0)
    out_ref[0] = h


def _deconv_stage(fc, pct, qp, ff, layers, q_tile):
    b, nc, cc = fc.shape
    q = qp.shape[1]
    q_tile = min(q_tile, q)
    d_out = layers[-1][0].shape[1]
    wargs = []
    for (w, ga, be) in layers:
        wargs += [w, ga.reshape(1, -1), be.reshape(1, -1)]
    has_ff = ff is not None
    grid = (b, q // q_tile)
    full = lambda arr: pl.BlockSpec(arr.shape, lambda bi, qi: (0,) * arr.ndim)
    in_specs = [
        pl.BlockSpec((1, nc, cc), lambda bi, qi: (bi, 0, 0)),
        pl.BlockSpec((1, 3, nc), lambda bi, qi: (bi, 0, 0)),
        pl.BlockSpec((1, q_tile, 3), lambda bi, qi: (bi, qi, 0)),
    ]
    args = [fc, pct, qp]
    if has_ff:
        in_specs.append(pl.BlockSpec((1, q_tile, ff.shape[2]),
                                     lambda bi, qi: (bi, qi, 0)))
        args.append(ff)
    in_specs += [full(a) for a in wargs]
    args += wargs
    return pl.pallas_call(
        functools.partial(_deconv_kernel, n_layers=len(layers), has_ff=has_ff),
        grid=grid,
        in_specs=in_specs,
        out_specs=pl.BlockSpec((1, q_tile, d_out), lambda bi, qi: (bi, qi, 0)),
        out_shape=jax.ShapeDtypeStruct((b, q, d_out), jnp.float32),
    )(*args)


def _head_kernel(f_ref, *wrefs, out_ref, n_layers):
    layers = [(wrefs[3 * i][...], wrefs[3 * i + 1][...], wrefs[3 * i + 2][...])
              for i in range(n_layers)]
    fcw = wrefs[3 * n_layers][...]
    fcb = wrefs[3 * n_layers + 1][...]
    h = f_ref[0]
    for w, ga, be in layers:
        h = jnp.maximum(_dot(h, w) * ga + be, 0.0)
    out_ref[0] = _dot(h, fcw) + fcb


def _head_stage(f, layers, fcw, fcb, q_tile):
    b, q, c = f.shape
    q_tile = min(q_tile, q)
    d_out = fcw.shape[1]
    wargs = []
    for (w, ga, be) in layers:
        wargs += [w, ga.reshape(1, -1), be.reshape(1, -1)]
    wargs += [fcw, fcb.reshape(1, -1)]
    full = lambda arr: pl.BlockSpec(arr.shape, lambda bi, qi: (0,) * arr.ndim)
    return pl.pallas_call(
        functools.partial(_head_kernel, n_layers=len(layers)),
        grid=(b, q // q_tile),
        in_specs=[pl.BlockSpec((1, q_tile, c), lambda bi, qi: (bi, qi, 0))]
                 + [full(a) for a in wargs],
        out_specs=pl.BlockSpec((1, q_tile, d_out), lambda bi, qi: (bi, qi, 0)),
        out_shape=jax.ShapeDtypeStruct((b, q, d_out), jnp.float32),
    )(f, *wargs)


def kernel(x, params):
    xt = jnp.transpose(x, (0, 2, 1))
    p0 = xt[..., :3]
    pct0 = x[:, :3]

    def ch(p):
        return jnp.transpose(p, (0, 2, 1))

    q1 = p0[:, ::4]
    f1 = _conv_stage(xt, pct0, q1, params['conv1'], 32, 256)
    s1 = jnp.concatenate([q1, f1], axis=-1)
    q2 = q1[:, ::4]
    f2 = _conv_stage(s1, ch(q1), q2, params['conv2'], 32, 256)
    s2 = jnp.concatenate([q2, f2], axis=-1)
    q3 = q2[:, ::4]
    f3 = _conv_stage(s2, ch(q2), q3, params['conv3'], 32, 64)
    s3 = jnp.concatenate([q3, f3], axis=-1)
    q4 = q3[:, ::4]
    f4 = _conv_stage(s3, ch(q3), q4, params['conv4'], 32, 16)

    g3 = _deconv_stage(f4, ch(q4), q3, f3, params['dconv1'], 64)
    g2 = _deconv_stage(g3, ch(q3), q2, f2, params['dconv2'], 256)
    g1 = _deconv_stage(g2, ch(q2), q1, f1, params['dconv3'], 512)
    g0 = _deconv_stage(g1, ch(q1), p0, None, params['dconv4'], 512)

    out = _head_stage(g0, params['mlp'], params['fc_w'], params['fc_b'], 1024)
    return jnp.transpose(out, (0, 2, 1))


# speedup vs baseline: 3.1977x; 3.1977x over previous
import functools

import jax
import jax.numpy as jnp
from jax.experimental import pallas as pl

_HI = jax.lax.Precision.HIGHEST
_BIG = 3.0e38


def _dot(a, b):
    return jax.lax.dot_general(a, b, (((1,), (0,)), ((), ())),
                               precision=_HI, preferred_element_type=jnp.float32)


def _dotd(a, b):
    return jax.lax.dot_general(a.astype(jnp.bfloat16), b.astype(jnp.bfloat16),
                               (((1,), (0,)), ((), ())),
                               preferred_element_type=jnp.float32)


def _dist_matrix(qp, pct):
    q2 = jnp.sum(qp * qp, axis=1, keepdims=True)
    p2 = jnp.sum(pct * pct, axis=0, keepdims=True)
    cross = _dotd(qp, pct)
    return q2 + p2 - 2.0 * cross


def _argmin_onehot(dist, iota):
    m = jnp.min(dist, axis=1, keepdims=True)
    cand = jnp.where(dist <= m, iota, jnp.int32(0x7FFFFFFF))
    am = jnp.min(cand, axis=1, keepdims=True)
    return m, iota == am


def _conv_kernel(s_ref, pct_ref, qp_ref, *wrefs, n_layers, k_nn):
    wrefs, out_ref = wrefs[:-1], wrefs[-1]
    layers = [(wrefs[3 * i][...], wrefs[3 * i + 1][...], wrefs[3 * i + 2][...])
              for i in range(n_layers)]
    s = s_ref[0]
    pct = pct_ref[0]
    qp = qp_ref[0]
    n = s.shape[0]
    qt = qp.shape[0]
    d_out = layers[-1][0].shape[1]

    dist = _dist_matrix(qp, pct)
    iota = jax.lax.broadcasted_iota(jnp.int32, (qt, n), 1)
    w1 = layers[0][0]

    def body(_, carry):
        dist, acc = carry
        _, onehot = _argmin_onehot(dist, iota)
        g = _dot(onehot.astype(jnp.float32), s)
        g_cat = jnp.concatenate([g[:, :3] - qp, g[:, 3:]], axis=1)
        h = jnp.maximum(_dotd(g_cat, w1) * layers[0][1] + layers[0][2], 0.0)
        for w, ga, be in layers[1:]:
            h = jnp.maximum(_dotd(h, w) * ga + be, 0.0)
        acc = jnp.maximum(acc, h)
        dist = jnp.where(onehot, _BIG, dist)
        return dist, acc

    _, acc = jax.lax.fori_loop(
        0, k_nn, body, (dist, jnp.zeros((qt, d_out), jnp.float32)))
    out_ref[0] = acc


def _conv_stage(s, pct, qp, layers, k_nn, q_tile):
    b, n, d = s.shape
    q = qp.shape[1]
    q_tile = min(q_tile, q)
    d_out = layers[-1][0].shape[1]
    wargs = []
    for (w, ga, be) in layers:
        wargs += [w, ga.reshape(1, -1), be.reshape(1, -1)]
    grid = (b, q // q_tile)
    full = lambda arr: pl.BlockSpec(arr.shape, lambda bi, qi: (0,) * arr.ndim)
    return pl.pallas_call(
        functools.partial(_conv_kernel, n_layers=len(layers), k_nn=k_nn),
        grid=grid,
        in_specs=[
            pl.BlockSpec((1, n, d), lambda bi, qi: (bi, 0, 0)),
            pl.BlockSpec((1, 3, n), lambda bi, qi: (bi, 0, 0)),
            pl.BlockSpec((1, q_tile, 3), lambda bi, qi: (bi, qi, 0)),
        ] + [full(a) for a in wargs],
        out_specs=pl.BlockSpec((1, q_tile, d_out), lambda bi, qi: (bi, qi, 0)),
        out_shape=jax.ShapeDtypeStruct((b, q, d_out), jnp.float32),
    )(s, pct, qp, *wargs)


def _deconv_kernel(fc_ref, pct_ref, qp_ref, *wrefs, n_layers, has_ff):
    wrefs, out_ref = wrefs[:-1], wrefs[-1]
    off = 0
    if has_ff:
        ff = wrefs[0][0]
        off = 1
    layers = [(wrefs[off + 3 * i][...], wrefs[off + 3 * i + 1][...],
               wrefs[off + 3 * i + 2][...]) for i in range(n_layers)]
    fc = fc_ref[0]
    pct = pct_ref[0]
    qp = qp_ref[0]
    nc, cc = fc.shape
    qt = qp.shape[0]

    dist = _dist_matrix(qp, pct)
    iota = jax.lax.broadcasted_iota(jnp.int32, (qt, nc), 1)
    num = jnp.zeros((qt, cc), jnp.float32)
    den = jnp.zeros((qt, 1), jnp.float32)
    for _ in range(3):
        m, onehot = _argmin_onehot(dist, iota)
        f = _dot(onehot.astype(jnp.float32), fc)
        w = 1.0 / jnp.maximum(m, 1e-10)
        num = num + w * f
        den = den + w
        dist = jnp.where(onehot, _BIG, dist)
    interp = num / den

    w1, g1, b1 = layers[0]
    pre = _dotd(interp, w1[:cc, :])
    if has_ff:
        pre = pre + _dotd(ff, w1[cc:, :])
    h = jnp.maximum(pre * g1 + b1, 0.0)
    for w, ga, be in layers[1:]:
        h = jnp.maximum(_dotd(h, w) * ga + be, 0.0)
    out_ref[0] = h


def _deconv_stage(fc, pct, qp, ff, layers, q_tile):
    b, nc, cc = fc.shape
    q = qp.shape[1]
    q_tile = min(q_tile, q)
    d_out = layers[-1][0].shape[1]
    wargs = []
    for (w, ga, be) in layers:
        wargs += [w, ga.reshape(1, -1), be.reshape(1, -1)]
    has_ff = ff is not None
    grid = (b, q // q_tile)
    full = lambda arr: pl.BlockSpec(arr.shape, lambda bi, qi: (0,) * arr.ndim)
    in_specs = [
        pl.BlockSpec((1, nc, cc), lambda bi, qi: (bi, 0, 0)),
        pl.BlockSpec((1, 3, nc), lambda bi, qi: (bi, 0, 0)),
        pl.BlockSpec((1, q_tile, 3), lambda bi, qi: (bi, qi, 0)),
    ]
    args = [fc, pct, qp]
    if has_ff:
        in_specs.append(pl.BlockSpec((1, q_tile, ff.shape[2]),
                                     lambda bi, qi: (bi, qi, 0)))
        args.append(ff)
    in_specs += [full(a) for a in wargs]
    args += wargs
    return pl.pallas_call(
        functools.partial(_deconv_kernel, n_layers=len(layers), has_ff=has_ff),
        grid=grid,
        in_specs=in_specs,
        out_specs=pl.BlockSpec((1, q_tile, d_out), lambda bi, qi: (bi, qi, 0)),
        out_shape=jax.ShapeDtypeStruct((b, q, d_out), jnp.float32),
    )(*args)


def _head_kernel(f_ref, *wrefs, n_layers):
    wrefs, out_ref = wrefs[:-1], wrefs[-1]
    layers = [(wrefs[3 * i][...], wrefs[3 * i + 1][...], wrefs[3 * i + 2][...])
              for i in range(n_layers)]
    fcw = wrefs[3 * n_layers][...]
    fcb = wrefs[3 * n_layers + 1][...]
    h = f_ref[0]
    for w, ga, be in layers:
        h = jnp.maximum(_dotd(h, w) * ga + be, 0.0)
    out_ref[0] = _dotd(h, fcw) + fcb


def _head_stage(f, layers, fcw, fcb, q_tile):
    b, q, c = f.shape
    q_tile = min(q_tile, q)
    d_out = fcw.shape[1]
    wargs = []
    for (w, ga, be) in layers:
        wargs += [w, ga.reshape(1, -1), be.reshape(1, -1)]
    wargs += [fcw, fcb.reshape(1, -1)]
    full = lambda arr: pl.BlockSpec(arr.shape, lambda bi, qi: (0,) * arr.ndim)
    return pl.pallas_call(
        functools.partial(_head_kernel, n_layers=len(layers)),
        grid=(b, q // q_tile),
        in_specs=[pl.BlockSpec((1, q_tile, c), lambda bi, qi: (bi, qi, 0))]
                 + [full(a) for a in wargs],
        out_specs=pl.BlockSpec((1, q_tile, d_out), lambda bi, qi: (bi, qi, 0)),
        out_shape=jax.ShapeDtypeStruct((b, q, d_out), jnp.float32),
    )(f, *wargs)


def kernel(x, params):
    xt = jnp.transpose(x, (0, 2, 1))
    p0 = xt[..., :3]
    pct0 = x[:, :3]

    def ch(p):
        return jnp.transpose(p, (0, 2, 1))

    q1 = p0[:, ::4]
    f1 = _conv_stage(xt, pct0, q1, params['conv1'], 32, 256)
    s1 = jnp.concatenate([q1, f1], axis=-1)
    q2 = q1[:, ::4]
    f2 = _conv_stage(s1, ch(q1), q2, params['conv2'], 32, 256)
    s2 = jnp.concatenate([q2, f2], axis=-1)
    q3 = q2[:, ::4]
    f3 = _conv_stage(s2, ch(q2), q3, params['conv3'], 32, 64)
    s3 = jnp.concatenate([q3, f3], axis=-1)
    q4 = q3[:, ::4]
    f4 = _conv_stage(s3, ch(q3), q4, params['conv4'], 32, 16)

    g3 = _deconv_stage(f4, ch(q4), q3, f3, params['dconv1'], 64)
    g2 = _deconv_stage(g3, ch(q3), q2, f2, params['dconv2'], 256)
    g1 = _deconv_stage(g2, ch(q2), q1, f1, params['dconv3'], 512)
    g0 = _deconv_stage(g1, ch(q1), p0, None, params['dconv4'], 512)

    out = _head_stage(g0, params['mlp'], params['fc_w'], params['fc_b'], 1024)
    return jnp.transpose(out, (0, 2, 1))


# speedup vs baseline: 5.3407x; 1.6702x over previous
import functools

import jax
import jax.numpy as jnp
from jax.experimental import pallas as pl

_HI = jax.lax.Precision.HIGHEST
_BIG = 3.0e38


def _dot(a, b):
    return jax.lax.dot_general(a, b, (((1,), (0,)), ((), ())),
                               precision=_HI, preferred_element_type=jnp.float32)


def _bsplit(b):
    bh = b.astype(jnp.bfloat16)
    r = b - bh.astype(jnp.float32)
    bm = r.astype(jnp.bfloat16)
    bl = (r - bm.astype(jnp.float32)).astype(jnp.bfloat16)
    return bh, bm, bl


def _dot2(a, parts):
    ab = a.astype(jnp.bfloat16)
    dims = (((1,), (0,)), ((), ()))
    out = None
    for p in parts:
        d = jax.lax.dot_general(ab, p, dims, preferred_element_type=jnp.float32)
        out = d if out is None else out + d
    return out


def _dotd(a, b):
    return jax.lax.dot_general(a.astype(jnp.bfloat16), b.astype(jnp.bfloat16),
                               (((1,), (0,)), ((), ())),
                               preferred_element_type=jnp.float32)


def _dist_matrix(qp, pct):
    q2 = jnp.sum(qp * qp, axis=1, keepdims=True)
    p2 = jnp.sum(pct * pct, axis=0, keepdims=True)
    cross = _dotd(qp, pct)
    return q2 + p2 - 2.0 * cross


def _argmin_onehot(dist, iota):
    m = jnp.min(dist, axis=1, keepdims=True)
    cand = jnp.where(dist <= m, iota, jnp.int32(0x7FFFFFFF))
    am = jnp.min(cand, axis=1, keepdims=True)
    return m, am, iota == am


def _conv_kernel(s_ref, pct_ref, qp_ref, *wrefs, n_layers, k_nn, d_feat, fold):
    wrefs, out_ref = wrefs[:-1], wrefs[-1]
    layers = [(wrefs[3 * i][...], wrefs[3 * i + 1][...], wrefs[3 * i + 2][...])
              for i in range(n_layers)]
    s = s_ref[0]
    pct = pct_ref[0]
    qp = qp_ref[0]
    n = pct.shape[1]
    qt = qp.shape[0]
    d = d_feat
    d_out = layers[-1][0].shape[1]

    dist = _dist_matrix(qp, pct)
    iota = jax.lax.broadcasted_iota(jnp.int32, (qt, n), 1)
    if fold:
        iota_c = jax.lax.broadcasted_iota(jnp.int32, (qt, n // 8), 1)
    s_parts = _bsplit(s)
    w1 = layers[0][0]

    def body(_, carry):
        dist, acc = carry
        _, am, onehot = _argmin_onehot(dist, iota)
        if fold:
            rowhot = (iota_c == jax.lax.shift_right_logical(am, 3))
            t = _dot2(rowhot.astype(jnp.float32), s_parts)
            sub = jnp.bitwise_and(am, 7)
            g = t[:, 0:d]
            for j in range(1, 8):
                g = jnp.where(sub == j, t[:, j * d:(j + 1) * d], g)
        else:
            g = _dot2(onehot.astype(jnp.float32), s_parts)
        g_cat = jnp.concatenate([g[:, :3] - qp, g[:, 3:]], axis=1)
        h = jnp.maximum(_dotd(g_cat, w1) * layers[0][1] + layers[0][2], 0.0)
        for w, ga, be in layers[1:]:
            h = jnp.maximum(_dotd(h, w) * ga + be, 0.0)
        acc = jnp.maximum(acc, h)
        dist = jnp.where(onehot, _BIG, dist)
        return dist, acc

    _, acc = jax.lax.fori_loop(
        0, k_nn, body, (dist, jnp.zeros((qt, d_out), jnp.float32)))
    out_ref[0] = acc


def _conv_stage(s, pct, qp, layers, k_nn, q_tile):
    b, n, d = s.shape
    q = qp.shape[1]
    q_tile = min(q_tile, q)
    d_out = layers[-1][0].shape[1]
    fold = n >= 1024
    if fold:
        s = s.reshape(b, n // 8, 8 * d)
    wargs = []
    for (w, ga, be) in layers:
        wargs += [w, ga.reshape(1, -1), be.reshape(1, -1)]
    grid = (b, q // q_tile)
    full = lambda arr: pl.BlockSpec(arr.shape, lambda bi, qi: (0,) * arr.ndim)
    return pl.pallas_call(
        functools.partial(_conv_kernel, n_layers=len(layers), k_nn=k_nn,
                          d_feat=d, fold=fold),
        grid=grid,
        in_specs=[
            pl.BlockSpec((1,) + s.shape[1:], lambda bi, qi: (bi, 0, 0)),
            pl.BlockSpec((1, 3, n), lambda bi, qi: (bi, 0, 0)),
            pl.BlockSpec((1, q_tile, 3), lambda bi, qi: (bi, qi, 0)),
        ] + [full(a) for a in wargs],
        out_specs=pl.BlockSpec((1, q_tile, d_out), lambda bi, qi: (bi, qi, 0)),
        out_shape=jax.ShapeDtypeStruct((b, q, d_out), jnp.float32),
    )(s, pct, qp, *wargs)


def _deconv_kernel(fc_ref, pct_ref, qp_ref, *wrefs, n_layers, has_ff):
    wrefs, out_ref = wrefs[:-1], wrefs[-1]
    off = 0
    if has_ff:
        ff = wrefs[0][0]
        off = 1
    layers = [(wrefs[off + 3 * i][...], wrefs[off + 3 * i + 1][...],
               wrefs[off + 3 * i + 2][...]) for i in range(n_layers)]
    fc = fc_ref[0]
    pct = pct_ref[0]
    qp = qp_ref[0]
    nc, cc = fc.shape
    qt = qp.shape[0]

    dist = _dist_matrix(qp, pct)
    iota = jax.lax.broadcasted_iota(jnp.int32, (qt, nc), 1)
    fc_parts = _bsplit(fc)
    num = jnp.zeros((qt, cc), jnp.float32)
    den = jnp.zeros((qt, 1), jnp.float32)
    for _ in range(3):
        m, _, onehot = _argmin_onehot(dist, iota)
        f = _dot2(onehot.astype(jnp.float32), fc_parts)
        w = 1.0 / jnp.maximum(m, 1e-10)
        num = num + w * f
        den = den + w
        dist = jnp.where(onehot, _BIG, dist)
    interp = num / den

    w1, g1, b1 = layers[0]
    pre = _dotd(interp, w1[:cc, :])
    if has_ff:
        pre = pre + _dotd(ff, w1[cc:, :])
    h = jnp.maximum(pre * g1 + b1, 0.0)
    for w, ga, be in layers[1:]:
        h = jnp.maximum(_dotd(h, w) * ga + be, 0.0)
    out_ref[0] = h


def _deconv_stage(fc, pct, qp, ff, layers, q_tile):
    b, nc, cc = fc.shape
    q = qp.shape[1]
    q_tile = min(q_tile, q)
    d_out = layers[-1][0].shape[1]
    wargs = []
    for (w, ga, be) in layers:
        wargs += [w, ga.reshape(1, -1), be.reshape(1, -1)]
    has_ff = ff is not None
    grid = (b, q // q_tile)
    full = lambda arr: pl.BlockSpec(arr.shape, lambda bi, qi: (0,) * arr.ndim)
    in_specs = [
        pl.BlockSpec((1, nc, cc), lambda bi, qi: (bi, 0, 0)),
        pl.BlockSpec((1, 3, nc), lambda bi, qi: (bi, 0, 0)),
        pl.BlockSpec((1, q_tile, 3), lambda bi, qi: (bi, qi, 0)),
    ]
    args = [fc, pct, qp]
    if has_ff:
        in_specs.append(pl.BlockSpec((1, q_tile, ff.shape[2]),
                                     lambda bi, qi: (bi, qi, 0)))
        args.append(ff)
    in_specs += [full(a) for a in wargs]
    args += wargs
    return pl.pallas_call(
        functools.partial(_deconv_kernel, n_layers=len(layers), has_ff=has_ff),
        grid=grid,
        in_specs=in_specs,
        out_specs=pl.BlockSpec((1, q_tile, d_out), lambda bi, qi: (bi, qi, 0)),
        out_shape=jax.ShapeDtypeStruct((b, q, d_out), jnp.float32),
    )(*args)


def _head_kernel(f_ref, *wrefs, n_layers):
    wrefs, out_ref = wrefs[:-1], wrefs[-1]
    layers = [(wrefs[3 * i][...], wrefs[3 * i + 1][...], wrefs[3 * i + 2][...])
              for i in range(n_layers)]
    fcw = wrefs[3 * n_layers][...]
    fcb = wrefs[3 * n_layers + 1][...]
    h = f_ref[0]
    for w, ga, be in layers:
        h = jnp.maximum(_dotd(h, w) * ga + be, 0.0)
    out_ref[0] = _dotd(h, fcw) + fcb


def _head_stage(f, layers, fcw, fcb, q_tile):
    b, q, c = f.shape
    q_tile = min(q_tile, q)
    d_out = fcw.shape[1]
    wargs = []
    for (w, ga, be) in layers:
        wargs += [w, ga.reshape(1, -1), be.reshape(1, -1)]
    wargs += [fcw, fcb.reshape(1, -1)]
    full = lambda arr: pl.BlockSpec(arr.shape, lambda bi, qi: (0,) * arr.ndim)
    return pl.pallas_call(
        functools.partial(_head_kernel, n_layers=len(layers)),
        grid=(b, q // q_tile),
        in_specs=[pl.BlockSpec((1, q_tile, c), lambda bi, qi: (bi, qi, 0))]
                 + [full(a) for a in wargs],
        out_specs=pl.BlockSpec((1, q_tile, d_out), lambda bi, qi: (bi, qi, 0)),
        out_shape=jax.ShapeDtypeStruct((b, q, d_out), jnp.float32),
    )(f, *wargs)


def kernel(x, params):
    xt = jnp.transpose(x, (0, 2, 1))
    p0 = xt[..., :3]
    pct0 = x[:, :3]

    def ch(p):
        return jnp.transpose(p, (0, 2, 1))

    q1 = p0[:, ::4]
    f1 = _conv_stage(xt, pct0, q1, params['conv1'], 32, 256)
    s1 = jnp.concatenate([q1, f1], axis=-1)
    q2 = q1[:, ::4]
    f2 = _conv_stage(s1, ch(q1), q2, params['conv2'], 32, 256)
    s2 = jnp.concatenate([q2, f2], axis=-1)
    q3 = q2[:, ::4]
    f3 = _conv_stage(s2, ch(q2), q3, params['conv3'], 32, 64)
    s3 = jnp.concatenate([q3, f3], axis=-1)
    q4 = q3[:, ::4]
    f4 = _conv_stage(s3, ch(q3), q4, params['conv4'], 32, 16)

    g3 = _deconv_stage(f4, ch(q4), q3, f3, params['dconv1'], 64)
    g2 = _deconv_stage(g3, ch(q3), q2, f2, params['dconv2'], 256)
    g1 = _deconv_stage(g2, ch(q2), q1, f1, params['dconv3'], 512)
    g0 = _deconv_stage(g1, ch(q1), p0, None, params['dconv4'], 512)

    out = _head_stage(g0, params['mlp'], params['fc_w'], params['fc_b'], 1024)
    return jnp.transpose(out, (0, 2, 1))


# speedup vs baseline: 7.2233x; 1.3525x over previous
import functools

import jax
import jax.numpy as jnp
from jax.experimental import pallas as pl

_HI = jax.lax.Precision.HIGHEST
_BIG = 3.0e38


def _dot(a, b):
    return jax.lax.dot_general(a, b, (((1,), (0,)), ((), ())),
                               precision=_HI, preferred_element_type=jnp.float32)


def _bsplit(b):
    bh = b.astype(jnp.bfloat16)
    r = b - bh.astype(jnp.float32)
    bm = r.astype(jnp.bfloat16)
    bl = (r - bm.astype(jnp.float32)).astype(jnp.bfloat16)
    return bh, bm, bl


def _dot2(a, parts):
    ab = a.astype(jnp.bfloat16)
    dims = (((1,), (0,)), ((), ()))
    out = None
    for p in parts:
        d = jax.lax.dot_general(ab, p, dims, preferred_element_type=jnp.float32)
        out = d if out is None else out + d
    return out


def _dotd(a, b):
    return jax.lax.dot_general(a.astype(jnp.bfloat16), b.astype(jnp.bfloat16),
                               (((1,), (0,)), ((), ())),
                               preferred_element_type=jnp.float32)


def _dist_matrix(qp, pct):
    q2 = jnp.sum(qp * qp, axis=1, keepdims=True)
    p2 = jnp.sum(pct * pct, axis=0, keepdims=True)
    cross = _dotd(qp, pct)
    return q2 + p2 - 2.0 * cross


def _argmin(dist, iota):
    m = jnp.min(dist, axis=1, keepdims=True)
    cand = jnp.where(dist <= m, iota, jnp.int32(0x7FFFFFFF))
    am = jnp.min(cand, axis=1, keepdims=True)
    return m, am


def _conv_kernel(s_ref, pct_ref, qp_ref, *wrefs, n_layers, k_nn, d_feat, fold):
    wrefs, out_ref = wrefs[:-1], wrefs[-1]
    layers = [(wrefs[3 * i][...], wrefs[3 * i + 1][...], wrefs[3 * i + 2][...])
              for i in range(n_layers)]
    s = s_ref[0]
    pct = pct_ref[0]
    qp = qp_ref[0]
    n = pct.shape[1]
    qt = qp.shape[0]
    d = d_feat
    d_out = layers[-1][0].shape[1]

    dist = _dist_matrix(qp, pct)
    iota = jax.lax.broadcasted_iota(jnp.int32, (qt, n), 1)
    if fold:
        iota_c = jax.lax.broadcasted_iota(jnp.int32, (qt, n // 8), 1)
    s_parts = _bsplit(s)
    w1 = layers[0][0]

    def body(_, carry):
        dist, acc = carry
        _, am = _argmin(dist, iota)
        if fold:
            rowhot = (iota_c == jax.lax.shift_right_logical(am, 3))
            t = _dot2(rowhot.astype(jnp.float32), s_parts)
            sub = jnp.bitwise_and(am, 7)
            g = t[:, 0:d]
            for j in range(1, 8):
                g = jnp.where(sub == j, t[:, j * d:(j + 1) * d], g)
        else:
            g = _dot2((iota == am).astype(jnp.float32), s_parts)
        g_cat = jnp.concatenate([g[:, :3] - qp, g[:, 3:]], axis=1)
        h = jnp.maximum(_dotd(g_cat, w1) * layers[0][1] + layers[0][2], 0.0)
        for w, ga, be in layers[1:]:
            h = jnp.maximum(_dotd(h, w) * ga + be, 0.0)
        acc = jnp.maximum(acc, h)
        dist = jnp.where(iota == am, _BIG, dist)
        return dist, acc

    _, acc = jax.lax.fori_loop(
        0, k_nn, body, (dist, jnp.zeros((qt, d_out), jnp.float32)),
        unroll=2)
    out_ref[0] = acc


def _conv_stage(s, pct, qp, layers, k_nn, q_tile):
    b, n, d = s.shape
    q = qp.shape[1]
    q_tile = min(q_tile, q)
    d_out = layers[-1][0].shape[1]
    fold = n >= 1024
    if fold:
        s = s.reshape(b, n // 8, 8 * d)
    wargs = []
    for (w, ga, be) in layers:
        wargs += [w, ga.reshape(1, -1), be.reshape(1, -1)]
    grid = (b, q // q_tile)
    full = lambda arr: pl.BlockSpec(arr.shape, lambda bi, qi: (0,) * arr.ndim)
    return pl.pallas_call(
        functools.partial(_conv_kernel, n_layers=len(layers), k_nn=k_nn,
                          d_feat=d, fold=fold),
        grid=grid,
        in_specs=[
            pl.BlockSpec((1,) + s.shape[1:], lambda bi, qi: (bi, 0, 0)),
            pl.BlockSpec((1, 3, n), lambda bi, qi: (bi, 0, 0)),
            pl.BlockSpec((1, q_tile, 3), lambda bi, qi: (bi, qi, 0)),
        ] + [full(a) for a in wargs],
        out_specs=pl.BlockSpec((1, q_tile, d_out), lambda bi, qi: (bi, qi, 0)),
        out_shape=jax.ShapeDtypeStruct((b, q, d_out), jnp.float32),
    )(s, pct, qp, *wargs)


def _deconv_kernel(fc_ref, pct_ref, qp_ref, *wrefs, n_layers, has_ff):
    wrefs, out_ref = wrefs[:-1], wrefs[-1]
    off = 0
    if has_ff:
        ff = wrefs[0][0]
        off = 1
    layers = [(wrefs[off + 3 * i][...], wrefs[off + 3 * i + 1][...],
               wrefs[off + 3 * i + 2][...]) for i in range(n_layers)]
    fc = fc_ref[0]
    pct = pct_ref[0]
    qp = qp_ref[0]
    nc, cc = fc.shape
    qt = qp.shape[0]

    dist = _dist_matrix(qp, pct)
    iota = jax.lax.broadcasted_iota(jnp.int32, (qt, nc), 1)
    fc_parts = _bsplit(fc)
    num = jnp.zeros((qt, cc), jnp.float32)
    den = jnp.zeros((qt, 1), jnp.float32)
    for _ in range(3):
        m, am = _argmin(dist, iota)
        f = _dot2((iota == am).astype(jnp.float32), fc_parts)
        w = 1.0 / jnp.maximum(m, 1e-10)
        num = num + w * f
        den = den + w
        dist = jnp.where(iota == am, _BIG, dist)
    interp = num / den

    w1, g1, b1 = layers[0]
    pre = _dotd(interp, w1[:cc, :])
    if has_ff:
        pre = pre + _dotd(ff, w1[cc:, :])
    h = jnp.maximum(pre * g1 + b1, 0.0)
    for w, ga, be in layers[1:]:
        h = jnp.maximum(_dotd(h, w) * ga + be, 0.0)
    out_ref[0] = h


def _deconv_stage(fc, pct, qp, ff, layers, q_tile):
    b, nc, cc = fc.shape
    q = qp.shape[1]
    q_tile = min(q_tile, q)
    d_out = layers[-1][0].shape[1]
    wargs = []
    for (w, ga, be) in layers:
        wargs += [w, ga.reshape(1, -1), be.reshape(1, -1)]
    has_ff = ff is not None
    grid = (b, q // q_tile)
    full = lambda arr: pl.BlockSpec(arr.shape, lambda bi, qi: (0,) * arr.ndim)
    in_specs = [
        pl.BlockSpec((1, nc, cc), lambda bi, qi: (bi, 0, 0)),
        pl.BlockSpec((1, 3, nc), lambda bi, qi: (bi, 0, 0)),
        pl.BlockSpec((1, q_tile, 3), lambda bi, qi: (bi, qi, 0)),
    ]
    args = [fc, pct, qp]
    if has_ff:
        in_specs.append(pl.BlockSpec((1, q_tile, ff.shape[2]),
                                     lambda bi, qi: (bi, qi, 0)))
        args.append(ff)
    in_specs += [full(a) for a in wargs]
    args += wargs
    return pl.pallas_call(
        functools.partial(_deconv_kernel, n_layers=len(layers), has_ff=has_ff),
        grid=grid,
        in_specs=in_specs,
        out_specs=pl.BlockSpec((1, q_tile, d_out), lambda bi, qi: (bi, qi, 0)),
        out_shape=jax.ShapeDtypeStruct((b, q, d_out), jnp.float32),
    )(*args)


def _head_kernel(f_ref, *wrefs, n_layers):
    wrefs, out_ref = wrefs[:-1], wrefs[-1]
    layers = [(wrefs[3 * i][...], wrefs[3 * i + 1][...], wrefs[3 * i + 2][...])
              for i in range(n_layers)]
    fcw = wrefs[3 * n_layers][...]
    fcb = wrefs[3 * n_layers + 1][...]
    h = f_ref[0]
    for w, ga, be in layers:
        h = jnp.maximum(_dotd(h, w) * ga + be, 0.0)
    out_ref[0] = _dotd(h, fcw) + fcb


def _head_stage(f, layers, fcw, fcb, q_tile):
    b, q, c = f.shape
    q_tile = min(q_tile, q)
    d_out = fcw.shape[1]
    wargs = []
    for (w, ga, be) in layers:
        wargs += [w, ga.reshape(1, -1), be.reshape(1, -1)]
    wargs += [fcw, fcb.reshape(1, -1)]
    full = lambda arr: pl.BlockSpec(arr.shape, lambda bi, qi: (0,) * arr.ndim)
    return pl.pallas_call(
        functools.partial(_head_kernel, n_layers=len(layers)),
        grid=(b, q // q_tile),
        in_specs=[pl.BlockSpec((1, q_tile, c), lambda bi, qi: (bi, qi, 0))]
                 + [full(a) for a in wargs],
        out_specs=pl.BlockSpec((1, q_tile, d_out), lambda bi, qi: (bi, qi, 0)),
        out_shape=jax.ShapeDtypeStruct((b, q, d_out), jnp.float32),
    )(f, *wargs)


def kernel(x, params):
    xt = jnp.transpose(x, (0, 2, 1))
    p0 = xt[..., :3]
    pct0 = x[:, :3]

    def ch(p):
        return jnp.transpose(p, (0, 2, 1))

    q1 = p0[:, ::4]
    f1 = _conv_stage(xt, pct0, q1, params['conv1'], 32, 512)
    s1 = jnp.concatenate([q1, f1], axis=-1)
    q2 = q1[:, ::4]
    f2 = _conv_stage(s1, ch(q1), q2, params['conv2'], 32, 256)
    s2 = jnp.concatenate([q2, f2], axis=-1)
    q3 = q2[:, ::4]
    f3 = _conv_stage(s2, ch(q2), q3, params['conv3'], 32, 64)
    s3 = jnp.concatenate([q3, f3], axis=-1)
    q4 = q3[:, ::4]
    f4 = _conv_stage(s3, ch(q3), q4, params['conv4'], 32, 16)

    g3 = _deconv_stage(f4, ch(q4), q3, f3, params['dconv1'], 64)
    g2 = _deconv_stage(g3, ch(q3), q2, f2, params['dconv2'], 256)
    g1 = _deconv_stage(g2, ch(q2), q1, f1, params['dconv3'], 1024)
    g0 = _deconv_stage(g1, ch(q1), p0, None, params['dconv4'], 1024)

    out = _head_stage(g0, params['mlp'], params['fc_w'], params['fc_b'], 1024)
    return jnp.transpose(out, (0, 2, 1))
